# baseline (speedup 1.0000x reference)
import functools

import jax
import jax.numpy as jnp
from jax import lax
from jax.experimental import pallas as pl
from jax.experimental.pallas import tpu as pltpu
from jax.experimental.pallas import tpu_sc as plsc

B = 256
L = 32
NC = 2
NS = 16
LANES = 16
BPW = 8
CH_P = 8
NBUF = 3

_CHUNKS = []
for _l in range(1, L):
    _np = L - _l
    _l8 = -(-_l // 8) * 8
    for _p0 in range(0, _np, CH_P):
        _CHUNKS.append((_l, _p0, min(CH_P, _np - _p0), _l8))


def _cky_body(scalars_hbm, out_hbm, chart, stg0, stg1, stg2, res, sem):
    wid = lax.axis_index("s") * NC + lax.axis_index("c")

    if True:
        b0 = wid * BPW
        stages = (stg0, stg1, stg2)

        bt0 = pl.multiple_of((wid // 16) * 128, 128)
        lane0 = (wid % 16) * BPW

        def issue(k):
            l, p0, plen, l8 = _CHUNKS[k]
            if l <= 12:
                return [
                    pltpu.async_copy(
                        scalars_hbm.at[
                            l, pl.ds(p0, plen), pl.ds(j, 8), pl.ds(b0, BPW)
                        ],
                        stages[k % NBUF].at[
                            pl.ds(0, plen), pl.ds(j, 8), pl.ds(0, BPW)
                        ],
                        sem,
                    )
                    for j in range(0, l8, 8)
                ]
            return [
                pltpu.async_copy(
                    scalars_hbm.at[
                        l, pl.ds(p0, plen), pl.ds(0, l8), pl.ds(bt0, 128)
                    ],
                    stages[k % NBUF].at[pl.ds(0, plen), pl.ds(0, l8), :],
                    sem,
                )
            ]

        ones = jnp.ones((LANES,), jnp.float32)

        def init_p(p, _):
            chart[0, pl.ds(p * LANES, LANES)] = ones
            return 0

        lax.fori_loop(0, L, init_p, 0)

        pending = [issue(k) for k in range(NBUF)]

        for k, (l, p0, plen, l8) in enumerate(_CHUNKS):
            for c in pending[k % NBUF]:
                c.wait()
            stg = stages[k % NBUF]

            ln0 = 0 if l <= 12 else lane0

            def cell(p_local, _, stg=stg, lvl=l, p0=p0, ln0=ln0):
                p = p0 + p_local

                def split(i, m):
                    x = chart[i, pl.ds(p * LANES, LANES)]
                    y = chart[lvl - 1 - i, pl.ds((p + i + 1) * LANES, LANES)]
                    s = stg[p_local, i, pl.ds(ln0, LANES)]
                    return jnp.maximum(m, x + y + s)

                m = lax.fori_loop(
                    0, lvl, split, jnp.full((LANES,), -jnp.inf, jnp.float32)
                )
                chart[lvl, pl.ds(p * LANES, LANES)] = m
                return 0

            lax.fori_loop(0, plen, cell, 0)
            if k + NBUF < len(_CHUNKS):
                pending[k % NBUF] = issue(k + NBUF)

        res[pl.ds(0, LANES)] = chart[L - 1, pl.ds(0, LANES)]
        pltpu.sync_copy(res.at[pl.ds(0, BPW)], out_hbm.at[pl.ds(b0, BPW)])


@jax.jit
def _cky_call(scalars_t):
    mesh = plsc.VectorSubcoreMesh(
        core_axis_name="c", subcore_axis_name="s", num_cores=NC, num_subcores=NS
    )
    return pl.kernel(
        _cky_body,
        out_type=jax.ShapeDtypeStruct((B,), jnp.float32),
        mesh=mesh,
        compiler_params=pltpu.CompilerParams(needs_layout_passes=False),
        scratch_types=[
            pltpu.VMEM((L, L * LANES), jnp.float32),
            pltpu.VMEM((CH_P, L, 128), jnp.float32),
            pltpu.VMEM((CH_P, L, 128), jnp.float32),
            pltpu.VMEM((CH_P, L, 128), jnp.float32),
            pltpu.VMEM((LANES,), jnp.float32),
            pltpu.SemaphoreType.DMA,
        ],
    )(scalars_t)


def kernel(sentences, scalars):
    del sentences
    return _cky_call(jnp.transpose(scalars, (0, 1, 3, 2)))


# speedup vs baseline: 1.1602x; 1.1602x over previous
import jax
import jax.numpy as jnp
from jax import lax
from jax.experimental import pallas as pl
from jax.experimental.pallas import tpu as pltpu
from jax.experimental.pallas import tpu_sc as plsc

B = 256
L = 32
NC = 2
NS = 16
LANES = 16
NG = B // LANES
CH_P = 8
NBUF = 3

_CHUNKS = []
for _l in range(1, L):
    _np = L - _l
    _l8 = -(-_l // 8) * 8
    for _p0 in range(0, _np, CH_P):
        _CHUNKS.append((_l, _p0, min(CH_P, _np - _p0), _l8))


def _cky_body(scalars_hbm, out_hbm, chart, stg0, stg1, stg2, res, sem):
    wid = lax.axis_index("s") * NC + lax.axis_index("c")

    @pl.when(wid < NG)
    def _():
        b0 = wid * LANES
        stages = (stg0, stg1, stg2)

        bt0 = pl.multiple_of((wid // 8) * 128, 128)
        lane0 = (wid % 8) * LANES

        def issue(k):
            l, p0, plen, l8 = _CHUNKS[k]
            if l <= 12:
                return [
                    pltpu.async_copy(
                        scalars_hbm.at[
                            l, pl.ds(p0, plen), pl.ds(j, 8), pl.ds(b0, LANES)
                        ],
                        stages[k % NBUF].at[
                            pl.ds(0, plen), pl.ds(j, 8), pl.ds(0, LANES)
                        ],
                        sem,
                    )
                    for j in range(0, l8, 8)
                ]
            return [
                pltpu.async_copy(
                    scalars_hbm.at[
                        l, pl.ds(p0, plen), pl.ds(0, l8), pl.ds(bt0, 128)
                    ],
                    stages[k % NBUF].at[pl.ds(0, plen), pl.ds(0, l8), :],
                    sem,
                )
            ]

        ones = jnp.ones((LANES,), jnp.float32)

        def init_p(p, _):
            chart[0, pl.ds(p * LANES, LANES)] = ones
            return 0

        lax.fori_loop(0, L, init_p, 0)

        pending = [issue(k) for k in range(NBUF)]

        for k, (l, p0, plen, l8) in enumerate(_CHUNKS):
            for c in pending[k % NBUF]:
                c.wait()
            stg = stages[k % NBUF]

            ln0 = 0 if l <= 12 else lane0

            def cell(p_local, _, stg=stg, lvl=l, p0=p0, ln0=ln0):
                p = p0 + p_local

                def split(i, m):
                    x = chart[i, pl.ds(p * LANES, LANES)]
                    y = chart[lvl - 1 - i, pl.ds((p + i + 1) * LANES, LANES)]
                    s = stg[p_local, i, pl.ds(ln0, LANES)]
                    return jnp.maximum(m, x + y + s)

                m = lax.fori_loop(
                    0, lvl, split, jnp.full((LANES,), -jnp.inf, jnp.float32)
                )
                chart[lvl, pl.ds(p * LANES, LANES)] = m
                return 0

            lax.fori_loop(0, plen, cell, 0)
            if k + NBUF < len(_CHUNKS):
                pending[k % NBUF] = issue(k + NBUF)

        res[pl.ds(0, LANES)] = chart[L - 1, pl.ds(0, LANES)]
        pltpu.sync_copy(res, out_hbm.at[pl.ds(b0, LANES)])


@jax.jit
def _cky_call(scalars_t):
    mesh = plsc.VectorSubcoreMesh(
        core_axis_name="c", subcore_axis_name="s", num_cores=NC, num_subcores=NS
    )
    return pl.kernel(
        _cky_body,
        out_type=jax.ShapeDtypeStruct((B,), jnp.float32),
        mesh=mesh,
        compiler_params=pltpu.CompilerParams(needs_layout_passes=False),
        scratch_types=[
            pltpu.VMEM((L, L * LANES), jnp.float32),
            pltpu.VMEM((CH_P, L, 128), jnp.float32),
            pltpu.VMEM((CH_P, L, 128), jnp.float32),
            pltpu.VMEM((CH_P, L, 128), jnp.float32),
            pltpu.VMEM((LANES,), jnp.float32),
            pltpu.SemaphoreType.DMA,
        ],
    )(scalars_t)


def kernel(sentences, scalars):
    del sentences
    return _cky_call(jnp.transpose(scalars, (0, 1, 3, 2)))
